# baseline (speedup 1.0000x reference)
import functools

import jax
import jax.numpy as jnp
from jax import lax
from jax.experimental import pallas as pl
from jax.experimental.pallas import tpu as pltpu
from jax.experimental.pallas import tpu_sc as plsc

_NC = 2
_NS = 16
_L = 16

_ROWS = 80
_LANES = 128


def _build_sc_call(num_entities, num_relations, num_edges):
    nw = _NC * _NS
    chunk = num_edges // nw
    groups = chunk // _L

    mesh = plsc.VectorSubcoreMesh(core_axis_name="c", subcore_axis_name="s")

    @functools.partial(
        pl.kernel,
        out_type=jax.ShapeDtypeStruct((_NC, _ROWS, _LANES), jnp.float32),
        mesh=mesh,
        compiler_params=pltpu.CompilerParams(needs_layout_passes=False),
        scratch_types=[
            pltpu.VMEM((chunk,), jnp.int32),
            pltpu.VMEM((chunk,), jnp.int32),
            pltpu.VMEM((chunk,), jnp.int32),
            pltpu.VMEM((num_entities,), jnp.float32),
            pltpu.VMEM((num_relations,), jnp.float32),
            pltpu.VMEM((_ROWS, _LANES), jnp.float32),
            pltpu.VMEM((_ROWS,), jnp.int32),
            pltpu.VMEM_SHARED((_ROWS, _LANES), jnp.float32),
            pltpu.SemaphoreType.DMA,
            pltpu.SemaphoreType.DMA,
        ],
    )
    def mp_kernel(idx_hbm, rel_hbm, ent_hbm, ps_hbm, out_hbm,
                  rows_v, cols_v, rel_v, ent_v, ps_v, acc_v, rowids_v,
                  shared_acc, sem0, sem1):
        cid = lax.axis_index("c")
        sid = lax.axis_index("s")
        wid = cid * _NS + sid
        base = wid * chunk
        half = chunk // 2

        copies0 = [
            pltpu.async_copy(idx_hbm.at[pl.ds(base, half)], rows_v.at[pl.ds(0, half)], sem0),
            pltpu.async_copy(idx_hbm.at[pl.ds(num_edges + base, half)],
                             cols_v.at[pl.ds(0, half)], sem0),
            pltpu.async_copy(rel_hbm.at[pl.ds(base, half)], rel_v.at[pl.ds(0, half)], sem0),
            pltpu.async_copy(ent_hbm, ent_v, sem0),
            pltpu.async_copy(ps_hbm, ps_v, sem0),
        ]
        copies1 = [
            pltpu.async_copy(idx_hbm.at[pl.ds(base + half, half)],
                             rows_v.at[pl.ds(half, half)], sem1),
            pltpu.async_copy(idx_hbm.at[pl.ds(num_edges + base + half, half)],
                             cols_v.at[pl.ds(half, half)], sem1),
            pltpu.async_copy(rel_hbm.at[pl.ds(base + half, half)],
                             rel_v.at[pl.ds(half, half)], sem1),
        ]

        zeros16 = jnp.zeros((_L,), jnp.float32)

        @plsc.parallel_loop(0, _ROWS, 1, unroll=2)
        def _(rrow):
            for kk in range(_LANES // _L):
                acc_v[rrow, pl.ds(kk * _L, _L)] = zeros16

        for kk in range(_ROWS // _L):
            rowids_v[pl.ds(kk * _L, _L)] = (
                lax.iota(jnp.int32, _L) + jnp.int32(kk * _L))

        @pl.when(sid == 0)
        def _():
            pltpu.sync_copy(acc_v, shared_acc)

        def edge_loop(g_lo, g_hi):
            @plsc.parallel_loop(g_lo, g_hi, 1, unroll=8)
            def _(i):
                off = i * _L
                r = rows_v[pl.ds(off, _L)]
                c = cols_v[pl.ds(off, _L)]
                m = rel_v[pl.ds(off, _L)]
                p = plsc.load_gather(ps_v, [m])
                e = plsc.load_gather(ent_v, [r])
                cf = c.astype(jnp.float32)
                q = (cf * jnp.float32(1.0 / num_entities)).astype(jnp.int32)
                j = c - q * num_entities
                j = jnp.where(j >= num_entities, j - num_entities, j)
                hi = lax.shift_right_logical(j, 7)
                lo = lax.bitwise_and(j, _LANES - 1)
                plsc.addupdate_scatter(acc_v, [hi, lo], p * e)

        for c in copies0:
            c.wait()
        edge_loop(0, groups // 2)
        for c in copies1:
            c.wait()
        edge_loop(groups // 2, groups)

        plsc.subcore_barrier()
        pltpu.sync_copy(acc_v, shared_acc.at[rowids_v], add=True)
        plsc.subcore_barrier()

        @pl.when(sid == 0)
        def _():
            pltpu.sync_copy(shared_acc, out_hbm.at[cid])

    return mp_kernel


def kernel(p_scores, indices, relation_mask, entities):
    num_entities = entities.shape[0]
    num_relations = p_scores.shape[0]
    num_edges = indices.shape[1]

    ent_flat = entities.reshape(num_entities)

    sc_call = _build_sc_call(num_entities, num_relations, num_edges)
    partials = sc_call(indices.reshape(2 * num_edges), relation_mask,
                       ent_flat, p_scores)
    y = partials.sum(axis=0).reshape(_ROWS * _LANES)[:num_entities]
    return (y, num_entities)


# speedup vs baseline: 1.0062x; 1.0062x over previous
import functools

import jax
import jax.numpy as jnp
from jax import lax
from jax.experimental import pallas as pl
from jax.experimental.pallas import tpu as pltpu
from jax.experimental.pallas import tpu_sc as plsc

_NC = 2
_NS = 16
_L = 16

_ROWS = 80
_LANES = 128


def _build_sc_call(num_entities, num_relations, num_edges):
    nw = _NC * _NS
    chunk = num_edges // nw
    groups = chunk // _L

    mesh = plsc.VectorSubcoreMesh(core_axis_name="c", subcore_axis_name="s")

    @functools.partial(
        pl.kernel,
        out_type=jax.ShapeDtypeStruct((_NC, _ROWS, _LANES), jnp.float32),
        mesh=mesh,
        compiler_params=pltpu.CompilerParams(needs_layout_passes=False),
        scratch_types=[
            pltpu.VMEM((chunk,), jnp.int32),
            pltpu.VMEM((chunk,), jnp.int32),
            pltpu.VMEM((chunk,), jnp.int32),
            pltpu.VMEM((num_entities,), jnp.float32),
            pltpu.VMEM((num_relations,), jnp.float32),
            pltpu.VMEM((_ROWS, _LANES), jnp.float32),
            pltpu.VMEM((_ROWS,), jnp.int32),
            pltpu.VMEM_SHARED((_ROWS, _LANES), jnp.float32),
            pltpu.SemaphoreType.DMA,
            pltpu.SemaphoreType.DMA,
        ],
    )
    def mp_kernel(idx_hbm, rel_hbm, ent_hbm, ps_hbm, out_hbm,
                  rows_v, cols_v, rel_v, ent_v, ps_v, acc_v, rowids_v,
                  shared_acc, sem0, sem1):
        cid = lax.axis_index("c")
        sid = lax.axis_index("s")
        wid = cid * _NS + sid
        base = wid * chunk
        half = chunk // 2

        copies0 = [
            pltpu.async_copy(idx_hbm.at[pl.ds(base, half)], rows_v.at[pl.ds(0, half)], sem0),
            pltpu.async_copy(idx_hbm.at[pl.ds(num_edges + base, half)],
                             cols_v.at[pl.ds(0, half)], sem0),
            pltpu.async_copy(rel_hbm.at[pl.ds(base, half)], rel_v.at[pl.ds(0, half)], sem0),
            pltpu.async_copy(ent_hbm, ent_v, sem0),
            pltpu.async_copy(ps_hbm, ps_v, sem0),
        ]
        copies1 = [
            pltpu.async_copy(idx_hbm.at[pl.ds(base + half, half)],
                             rows_v.at[pl.ds(half, half)], sem1),
            pltpu.async_copy(idx_hbm.at[pl.ds(num_edges + base + half, half)],
                             cols_v.at[pl.ds(half, half)], sem1),
            pltpu.async_copy(rel_hbm.at[pl.ds(base + half, half)],
                             rel_v.at[pl.ds(half, half)], sem1),
        ]

        zeros16 = jnp.zeros((_L,), jnp.float32)

        @plsc.parallel_loop(0, _ROWS, 1, unroll=2)
        def _(rrow):
            for kk in range(_LANES // _L):
                acc_v[rrow, pl.ds(kk * _L, _L)] = zeros16

        for kk in range(_ROWS // _L):
            rowids_v[pl.ds(kk * _L, _L)] = (
                lax.iota(jnp.int32, _L) + jnp.int32(kk * _L))

        @pl.when(sid == 0)
        def _():
            pltpu.sync_copy(acc_v, shared_acc)

        def edge_loop(g_lo, g_hi):
            @plsc.parallel_loop(g_lo, g_hi, 1, unroll=16)
            def _(i):
                off = i * _L
                r = rows_v[pl.ds(off, _L)]
                c = cols_v[pl.ds(off, _L)]
                m = rel_v[pl.ds(off, _L)]
                p = plsc.load_gather(ps_v, [m])
                e = plsc.load_gather(ent_v, [r])
                cf = c.astype(jnp.float32)
                q = (cf * jnp.float32(1.0 / num_entities)).astype(jnp.int32)
                j = c - q * num_entities
                j = jnp.where(j >= num_entities, j - num_entities, j)
                hi = lax.shift_right_logical(j, 7)
                lo = lax.bitwise_and(j, _LANES - 1)
                plsc.addupdate_scatter(acc_v, [hi, lo], p * e)

        for c in copies0:
            c.wait()
        edge_loop(0, groups // 2)
        for c in copies1:
            c.wait()
        edge_loop(groups // 2, groups)

        plsc.subcore_barrier()
        pltpu.sync_copy(acc_v, shared_acc.at[rowids_v], add=True)
        plsc.subcore_barrier()

        @pl.when(sid == 0)
        def _():
            pltpu.sync_copy(shared_acc, out_hbm.at[cid])

    return mp_kernel


def kernel(p_scores, indices, relation_mask, entities):
    num_entities = entities.shape[0]
    num_relations = p_scores.shape[0]
    num_edges = indices.shape[1]

    ent_flat = entities.reshape(num_entities)

    sc_call = _build_sc_call(num_entities, num_relations, num_edges)
    partials = sc_call(indices.reshape(2 * num_edges), relation_mask,
                       ent_flat, p_scores)
    y = partials.sum(axis=0).reshape(_ROWS * _LANES)[:num_entities]
    return (y, num_entities)
